# baseline (speedup 1.0000x reference)
import functools

import jax
import jax.numpy as jnp
from jax import lax
from jax.experimental import pallas as pl
from jax.experimental.pallas import tpu as pltpu
from jax.experimental.pallas import tpu_sc as plsc

NUM_FIELDS = 26
VOCAB = 100001
EMBED_DIM = 32
NUM_NUM = 13
BATCH = 16384
EPS = 1e-5

NC = 2
NS = 16
L = 16
FIELDS_PER_CORE = NUM_FIELDS // NC
ROWS_PER_TILE = BATCH // NS
GCHUNK = 128
NCHUNK = ROWS_PER_TILE // GCHUNK
PACK = 128 // EMBED_DIM
VBLOCKS = 196
S = VBLOCKS * 128
OUT_COLS = NUM_FIELDS * EMBED_DIM + NUM_NUM
OUT_PAD = 896


CH = S // 2


def _repack_body(t0, t1, t2, t3, out):
    out[:] = jnp.concatenate([t[0].T for t in (t0, t1, t2, t3)], axis=1)


@jax.jit
def _repack(tphys):
    specs = [
        pl.BlockSpec((1, EMBED_DIM, CH),
                     lambda f, c, q=q: (f, 0, q * 2 + c))
        for q in range(PACK)
    ]
    return pl.pallas_call(
        _repack_body,
        grid=(NUM_FIELDS, 2),
        in_specs=specs,
        out_specs=pl.BlockSpec((CH, 128), lambda f, c: (f * 2 + c, 0)),
        out_shape=jax.ShapeDtypeStruct((NUM_FIELDS * S, PACK * EMBED_DIM),
                                       jnp.float32),
    )(tphys, tphys, tphys, tphys)


def _rsqrt16(x):
    i = lax.bitcast_convert_type(x, jnp.int32)
    i = jnp.int32(0x5F3759DF) - lax.shift_right_logical(i, 1)
    y = lax.bitcast_convert_type(i, jnp.float32)
    for _ in range(3):
        y = y * (1.5 - 0.5 * x * y * y)
    return y


def _tile_body(cat_hbm, tbl_hbm, gam_hbm, bet_hbm, num_hbm, out_hbm,
               idxr, idxp, g, rows, partials, pall, gv, bv, numv,
               spmem, sems):
    c = lax.axis_index("c")
    s = lax.axis_index("s")
    row0 = s * ROWS_PER_TILE

    @pl.when(c == 0)
    def _():
        pltpu.sync_copy(num_hbm.at[pl.ds(row0, ROWS_PER_TILE)], numv)
        pltpu.sync_copy(
            numv,
            out_hbm.at[pl.ds(row0, ROWS_PER_TILE),
                       pl.ds(NUM_FIELDS * EMBED_DIM, L)])

    z = jnp.zeros((L,), jnp.float32)
    iota = lax.iota(jnp.int32, L)
    inv_s = jnp.float32(1.0 / S)

    def field_step(fl, carry):
        f = c * FIELDS_PER_CORE + fl

        pltpu.sync_copy(cat_hbm.at[f, pl.ds(s * NCHUNK, NCHUNK)], idxr)

        base = (f * S).astype(jnp.int32)

        def to_packed(j, _):
            for k in range(GCHUNK // L):
                v = idxr[j, pl.ds(k * L, L)]
                vf = v.astype(jnp.float32) + 0.5
                q = (vf * inv_s).astype(jnp.int32)
                idxp[j, pl.ds(k * L, L)] = base + v - q * S
                idxr[j, pl.ds(k * L, L)] = q * EMBED_DIM
            return 0

        lax.fori_loop(0, NCHUNK, to_packed, 0)

        waits = [
            pltpu.async_copy(tbl_hbm.at[idxp.at[0]], g.at[0], sems.at[0])
        ]
        for j in range(NCHUNK):
            if j + 1 < NCHUNK:
                b = (j + 1) % 2
                waits.append(pltpu.async_copy(
                    tbl_hbm.at[idxp.at[j + 1]], g.at[b], sems.at[b]))
            waits[j].wait()
            gb = g.at[j % 2]

            def extract16(t, _2, j=j, gb=gb):
                i0 = t * L
                iv = iota + i0
                voff = idxr[j, pl.ds(i0, L)]
                riv = j * GCHUNK + iv
                for d in range(EMBED_DIM):
                    x = plsc.load_gather(gb, [iv, voff + d])
                    plsc.store_scatter(rows, [riv, iota * 0 + d], x)
                return 0

            lax.fori_loop(0, GCHUNK // L, extract16, 0)

        def red_step(i, acc):
            s0, s1, q0, q1 = acc
            x0 = rows[i, pl.ds(0, L)]
            x1 = rows[i, pl.ds(L, L)]
            return (s0 + x0, s1 + x1, q0 + x0 * x0, q1 + x1 * x1)

        s0, s1, q0, q1 = lax.fori_loop(0, ROWS_PER_TILE, red_step,
                                       (z, z, z, z))
        partials[pl.ds(0, L)] = s0
        partials[pl.ds(L, L)] = s1
        partials[pl.ds(2 * L, L)] = q0
        partials[pl.ds(3 * L, L)] = q1

        pltpu.sync_copy(partials, spmem.at[fl, s])
        plsc.subcore_barrier()
        pltpu.sync_copy(spmem.at[fl], pall)

        def red16(r, acc):
            a0, a1, a2, a3 = acc
            return (a0 + pall[r, pl.ds(0, L)],
                    a1 + pall[r, pl.ds(L, L)],
                    a2 + pall[r, pl.ds(2 * L, L)],
                    a3 + pall[r, pl.ds(3 * L, L)])

        a0, a1, a2, a3 = lax.fori_loop(0, NS, red16, (z, z, z, z))
        inv_n = jnp.float32(1.0 / BATCH)
        m0 = a0 * inv_n
        m1 = a1 * inv_n
        v0 = a2 * inv_n - m0 * m0
        v1 = a3 * inv_n - m1 * m1
        r0 = _rsqrt16(v0 + EPS)
        r1 = _rsqrt16(v1 + EPS)

        pltpu.sync_copy(gam_hbm.at[pl.ds(f * EMBED_DIM, EMBED_DIM)], gv)
        pltpu.sync_copy(bet_hbm.at[pl.ds(f * EMBED_DIM, EMBED_DIM)], bv)
        sc0 = r0 * gv[pl.ds(0, L)]
        sc1 = r1 * gv[pl.ds(L, L)]
        sh0 = bv[pl.ds(0, L)] - m0 * sc0
        sh1 = bv[pl.ds(L, L)] - m1 * sc1

        def norm_step(i, _):
            x0 = rows[i, pl.ds(0, L)]
            x1 = rows[i, pl.ds(L, L)]
            rows[i, pl.ds(0, L)] = jnp.maximum(x0 * sc0 + sh0, 0.0)
            rows[i, pl.ds(L, L)] = jnp.maximum(x1 * sc1 + sh1, 0.0)
            return 0

        lax.fori_loop(0, ROWS_PER_TILE, norm_step, 0)

        pltpu.sync_copy(
            rows,
            out_hbm.at[pl.ds(row0, ROWS_PER_TILE),
                       pl.ds(EMBED_DIM * f, EMBED_DIM)])
        return carry

    lax.fori_loop(0, FIELDS_PER_CORE, field_step, 0)


@jax.jit
def _sc_call(cat_r, tbl, gam1, bet1, num16):
    mesh = plsc.VectorSubcoreMesh(core_axis_name="c", subcore_axis_name="s")
    return pl.kernel(
        _tile_body,
        out_type=jax.ShapeDtypeStruct((BATCH, OUT_PAD), jnp.float32),
        mesh=mesh,
        scratch_types=[
            pltpu.VMEM((NCHUNK, GCHUNK), jnp.int32),
            pltpu.VMEM((NCHUNK, GCHUNK), jnp.int32),
            pltpu.VMEM((2, GCHUNK, PACK * EMBED_DIM), jnp.float32),
            pltpu.VMEM((ROWS_PER_TILE, EMBED_DIM), jnp.float32),
            pltpu.VMEM((4 * L,), jnp.float32),
            pltpu.VMEM((NS, 4 * L), jnp.float32),
            pltpu.VMEM((EMBED_DIM,), jnp.float32),
            pltpu.VMEM((EMBED_DIM,), jnp.float32),
            pltpu.VMEM((ROWS_PER_TILE, L), jnp.float32),
            pltpu.VMEM_SHARED((FIELDS_PER_CORE, NS, 4 * L), jnp.float32),
            pltpu.SemaphoreType.DMA((2,)),
        ],
        compiler_params=pltpu.CompilerParams(use_tc_tiling_on_sc=False,
                                             needs_layout_passes=False),
        name="categorical_dnn_sc",
    )(cat_r, tbl, gam1, bet1, num16)


def kernel(input, emb_tables, gammas, betas):
    cat = input[:, :NUM_FIELDS].astype(jnp.int32)
    cat_r = cat.T.reshape(NUM_FIELDS, BATCH // GCHUNK, GCHUNK)
    num16 = jnp.pad(input[:, NUM_FIELDS:], ((0, 0), (0, L - NUM_NUM)))
    tphys = emb_tables.transpose(0, 2, 1)
    tbl = _repack(tphys)
    out = _sc_call(cat_r, tbl, gammas.reshape(-1), betas.reshape(-1), num16)
    return out[:, :OUT_COLS]


# speedup vs baseline: 1.4917x; 1.4917x over previous
import functools

import jax
import jax.numpy as jnp
from jax import lax
from jax.experimental import pallas as pl
from jax.experimental.pallas import tpu as pltpu
from jax.experimental.pallas import tpu_sc as plsc

NUM_FIELDS = 26
VOCAB = 100001
EMBED_DIM = 32
NUM_NUM = 13
BATCH = 16384
EPS = 1e-5

NC = 2
NS = 16
L = 16
FIELDS_PER_CORE = NUM_FIELDS // NC
ROWS_PER_TILE = BATCH // NS
GCHUNK = 128
NCHUNK = ROWS_PER_TILE // GCHUNK
PACK = 128 // EMBED_DIM
VBLOCKS = 196
S = VBLOCKS * 128
OUT_COLS = NUM_FIELDS * EMBED_DIM + NUM_NUM
OUT_PAD = 896


CH = S // 2


def _repack_body(t0, t1, t2, t3, out):
    out[:] = jnp.concatenate([t[0].T for t in (t0, t1, t2, t3)], axis=1)


@jax.jit
def _repack(tphys):
    specs = [
        pl.BlockSpec((1, EMBED_DIM, CH),
                     lambda f, c, q=q: (f, 0, q * 2 + c))
        for q in range(PACK)
    ]
    return pl.pallas_call(
        _repack_body,
        grid=(NUM_FIELDS, 2),
        in_specs=specs,
        out_specs=pl.BlockSpec((CH, 128), lambda f, c: (f * 2 + c, 0)),
        out_shape=jax.ShapeDtypeStruct((NUM_FIELDS * S, PACK * EMBED_DIM),
                                       jnp.float32),
    )(tphys, tphys, tphys, tphys)


def _rsqrt16(x):
    i = lax.bitcast_convert_type(x, jnp.int32)
    i = jnp.int32(0x5F3759DF) - lax.shift_right_logical(i, 1)
    y = lax.bitcast_convert_type(i, jnp.float32)
    for _ in range(3):
        y = y * (1.5 - 0.5 * x * y * y)
    return y


def _tile_body(cat_hbm, tbl_hbm, gam_hbm, bet_hbm, num_hbm, out_hbm,
               idxr, idxp, g, rows, partials, pall, gv, bv, numv,
               spmem, sems):
    c = lax.axis_index("c")
    s = lax.axis_index("s")
    row0 = s * ROWS_PER_TILE

    @pl.when(c == 0)
    def _():
        pltpu.sync_copy(num_hbm.at[pl.ds(row0, ROWS_PER_TILE)], numv)
        pltpu.sync_copy(
            numv,
            out_hbm.at[pl.ds(row0, ROWS_PER_TILE),
                       pl.ds(NUM_FIELDS * EMBED_DIM, L)])

    z = jnp.zeros((L,), jnp.float32)
    iota = lax.iota(jnp.int32, L)
    inv_s = jnp.float32(1.0 / S)

    def field_step(fl, carry):
        f = c * FIELDS_PER_CORE + fl

        pltpu.sync_copy(cat_hbm.at[f, pl.ds(s * NCHUNK, NCHUNK)], idxr)

        base = (f * S).astype(jnp.int32)

        def to_packed(j, _):
            for k in range(GCHUNK // L):
                v = idxr[j, pl.ds(k * L, L)]
                vf = v.astype(jnp.float32) + 0.5
                q = (vf * inv_s).astype(jnp.int32)
                idxp[j, pl.ds(k * L, L)] = base + v - q * S
                idxr[j, pl.ds(k * L, L)] = q * EMBED_DIM
            return 0

        lax.fori_loop(0, NCHUNK, to_packed, 0)

        waits = [
            pltpu.async_copy(tbl_hbm.at[idxp.at[0]], g.at[0], sems.at[0])
        ]
        for j in range(NCHUNK):
            if j + 1 < NCHUNK:
                b = (j + 1) % 3
                waits.append(pltpu.async_copy(
                    tbl_hbm.at[idxp.at[j + 1]], g.at[b], sems.at[b]))
            waits[j].wait()
            gb = g.at[j % 3]

            @functools.partial(plsc.parallel_loop, 0, GCHUNK // L, unroll=2)
            def extract16(t, j=j, gb=gb):
                i0 = t * L
                iv = iota + i0
                voff = idxr[j, pl.ds(i0, L)]
                riv = j * GCHUNK + iv
                for d in range(EMBED_DIM):
                    x = plsc.load_gather(gb, [iv, voff + d])
                    plsc.store_scatter(rows, [riv, iota * 0 + d], x)

        def red_step(i, acc):
            s0, s1, q0, q1 = acc
            for u in range(2):
                x0 = rows[i * 2 + u, pl.ds(0, L)]
                x1 = rows[i * 2 + u, pl.ds(L, L)]
                s0 = s0 + x0
                s1 = s1 + x1
                q0 = q0 + x0 * x0
                q1 = q1 + x1 * x1
            return (s0, s1, q0, q1)

        s0, s1, q0, q1 = lax.fori_loop(0, ROWS_PER_TILE // 2, red_step,
                                       (z, z, z, z))
        partials[pl.ds(0, L)] = s0
        partials[pl.ds(L, L)] = s1
        partials[pl.ds(2 * L, L)] = q0
        partials[pl.ds(3 * L, L)] = q1

        pltpu.sync_copy(partials, spmem.at[fl, s])
        plsc.subcore_barrier()
        pltpu.sync_copy(spmem.at[fl], pall)

        def red16(r, acc):
            a0, a1, a2, a3 = acc
            return (a0 + pall[r, pl.ds(0, L)],
                    a1 + pall[r, pl.ds(L, L)],
                    a2 + pall[r, pl.ds(2 * L, L)],
                    a3 + pall[r, pl.ds(3 * L, L)])

        a0, a1, a2, a3 = lax.fori_loop(0, NS, red16, (z, z, z, z))
        inv_n = jnp.float32(1.0 / BATCH)
        m0 = a0 * inv_n
        m1 = a1 * inv_n
        v0 = a2 * inv_n - m0 * m0
        v1 = a3 * inv_n - m1 * m1
        r0 = _rsqrt16(v0 + EPS)
        r1 = _rsqrt16(v1 + EPS)

        pltpu.sync_copy(gam_hbm.at[pl.ds(f * EMBED_DIM, EMBED_DIM)], gv)
        pltpu.sync_copy(bet_hbm.at[pl.ds(f * EMBED_DIM, EMBED_DIM)], bv)
        sc0 = r0 * gv[pl.ds(0, L)]
        sc1 = r1 * gv[pl.ds(L, L)]
        sh0 = bv[pl.ds(0, L)] - m0 * sc0
        sh1 = bv[pl.ds(L, L)] - m1 * sc1

        def norm_step(i, _):
            for u in range(2):
                x0 = rows[i * 2 + u, pl.ds(0, L)]
                x1 = rows[i * 2 + u, pl.ds(L, L)]
                rows[i * 2 + u, pl.ds(0, L)] = jnp.maximum(
                    x0 * sc0 + sh0, 0.0)
                rows[i * 2 + u, pl.ds(L, L)] = jnp.maximum(
                    x1 * sc1 + sh1, 0.0)
            return 0

        lax.fori_loop(0, ROWS_PER_TILE // 2, norm_step, 0)

        pltpu.sync_copy(
            rows,
            out_hbm.at[pl.ds(row0, ROWS_PER_TILE),
                       pl.ds(EMBED_DIM * f, EMBED_DIM)])
        return carry

    lax.fori_loop(0, FIELDS_PER_CORE, field_step, 0)


@jax.jit
def _sc_call(cat_r, tbl, gam1, bet1, num16):
    mesh = plsc.VectorSubcoreMesh(core_axis_name="c", subcore_axis_name="s")
    return pl.kernel(
        _tile_body,
        out_type=jax.ShapeDtypeStruct((BATCH, OUT_PAD), jnp.float32),
        mesh=mesh,
        scratch_types=[
            pltpu.VMEM((NCHUNK, GCHUNK), jnp.int32),
            pltpu.VMEM((NCHUNK, GCHUNK), jnp.int32),
            pltpu.VMEM((3, GCHUNK, PACK * EMBED_DIM), jnp.float32),
            pltpu.VMEM((ROWS_PER_TILE, EMBED_DIM), jnp.float32),
            pltpu.VMEM((4 * L,), jnp.float32),
            pltpu.VMEM((NS, 4 * L), jnp.float32),
            pltpu.VMEM((EMBED_DIM,), jnp.float32),
            pltpu.VMEM((EMBED_DIM,), jnp.float32),
            pltpu.VMEM((ROWS_PER_TILE, L), jnp.float32),
            pltpu.VMEM_SHARED((FIELDS_PER_CORE, NS, 4 * L), jnp.float32),
            pltpu.SemaphoreType.DMA((3,)),
        ],
        compiler_params=pltpu.CompilerParams(use_tc_tiling_on_sc=False,
                                             needs_layout_passes=False),
        name="categorical_dnn_sc",
    )(cat_r, tbl, gam1, bet1, num16)


def kernel(input, emb_tables, gammas, betas):
    cat = input[:, :NUM_FIELDS].astype(jnp.int32)
    cat_r = cat.T.reshape(NUM_FIELDS, BATCH // GCHUNK, GCHUNK)
    num16 = jnp.pad(input[:, NUM_FIELDS:], ((0, 0), (0, L - NUM_NUM)))
    tphys = emb_tables.transpose(0, 2, 1)
    tbl = _repack(tphys)
    out = _sc_call(cat_r, tbl, gammas.reshape(-1), betas.reshape(-1), num16)
    return out[:, :OUT_COLS]
